# baseline (speedup 1.0000x reference)
import functools

import jax
import jax.numpy as jnp
from jax import lax
from jax.experimental import pallas as pl
from jax.experimental.pallas import tpu as pltpu
from jax.experimental.pallas import tpu_sc as plsc

N, E, D, ED, H, C, G = 10000, 320000, 128, 16, 128, 10, 128
BN_EPS = 1e-5
NP = 10240
NC, NS, L = 2, 16, 16
NW = NC * NS
EPW = E // NW
CE = 40
NCHUNK = EPW // CE
ROWS_PT = NP // NS

_EBLK = 8000


def _edge_body(ea, We, be, e):
    e[:] = jnp.dot(ea[:], We[:], preferred_element_type=jnp.float32) + be[:]


def _edge_transform(ea, We, be):
    nblk = E // _EBLK
    return pl.pallas_call(
        _edge_body,
        grid=(nblk,),
        in_specs=[pl.BlockSpec((_EBLK, ED), lambda i: (i, 0)),
                  pl.BlockSpec((ED, D), lambda i: (0, 0)),
                  pl.BlockSpec((1, D), lambda i: (0, 0))],
        out_specs=pl.BlockSpec((_EBLK, D), lambda i: (i, 0)),
        out_shape=jax.ShapeDtypeStruct((E, D), jnp.float32),
    )(ea, We, be)


_sc_mesh = plsc.VectorSubcoreMesh(
    core_axis_name="c", subcore_axis_name="s", num_cores=NC, num_subcores=NS)


@functools.partial(
    pl.kernel,
    out_type=jax.ShapeDtypeStruct((NC, NP, D), jnp.float32),
    mesh=_sc_mesh,
    scratch_types=[
        pltpu.VMEM((8, 2, CE), jnp.int32),
        pltpu.VMEM((4, CE, D), jnp.float32),
        pltpu.VMEM((2, CE, D), jnp.float32),
        pltpu.VMEM_SHARED((NP, D), jnp.float32),
        pltpu.SemaphoreType.DMA,
        pltpu.SemaphoreType.DMA,
        pltpu.SemaphoreType.DMA,
        pltpu.SemaphoreType.DMA,
        pltpu.SemaphoreType.DMA,
        pltpu.SemaphoreType.DMA,
        pltpu.SemaphoreType.DMA,
        pltpu.SemaphoreType.DMA,
        pltpu.SemaphoreType.DMA,
        pltpu.SemaphoreType.DMA,
        pltpu.SemaphoreType.DMA,
        pltpu.SemaphoreType.DMA,
        pltpu.SemaphoreType.DMA,
        pltpu.SemaphoreType.DMA,
    ],
)
def _sc_msg(x_hbm, e_hbm, idx_hbm, zero_hbm, out_hbm,
            idxb, xbuf, ebuf, agg,
            i0, i1, i2, i3, g0, g1, g2, g3, m0, m1, s0, s1, s2, s3):
    c = lax.axis_index("c")
    s = lax.axis_index("s")
    wid = s * NC + c
    isem = (i0, i1, i2, i3)
    gsem = (g0, g1, g2, g3)
    msem = (m0, m1)
    ssem = (s0, s1, s2, s3)

    @pl.when(s == 0)
    def _():
        pltpu.sync_copy(zero_hbm, agg)

    plsc.subcore_barrier()

    def issue_idx(j, o):
        pltpu.async_copy(idx_hbm.at[wid, j], idxb.at[o % 8], isem[o % 4])

    def wait_idx(j, o):
        pltpu.make_async_copy(idx_hbm.at[wid, j], idxb.at[o % 8],
                              isem[o % 4]).wait()

    def issue_fetch(j, o):
        pltpu.async_copy(x_hbm.at[idxb.at[o % 8, 0]], xbuf.at[o % 4],
                         gsem[o % 4])

    def issue_emsg(j, o):
        ebase = wid * EPW + j * CE
        pltpu.async_copy(e_hbm.at[pl.ds(ebase, CE), :], ebuf.at[o % 2],
                         msem[o % 2])

    def wait_fetch(j, o):
        pltpu.make_async_copy(x_hbm.at[idxb.at[o % 8, 0]], xbuf.at[o % 4],
                              gsem[o % 4]).wait()
        ebase = wid * EPW + j * CE
        pltpu.make_async_copy(e_hbm.at[pl.ds(ebase, CE), :], ebuf.at[o % 2],
                              msem[o % 2]).wait()

    def issue_scatter(j, o):
        pltpu.async_copy(xbuf.at[o % 4], agg.at[idxb.at[o % 8, 1]],
                         ssem[o % 4], add=True)

    def wait_scatter(j, o):
        pltpu.make_async_copy(xbuf.at[o % 4], agg.at[idxb.at[o % 8, 1]],
                              ssem[o % 4]).wait()

    def step(i, o, pf_ft, pf_ix, guarded=True):
        wait_fetch(i, o)
        if guarded:
            @pl.when(i >= 2)
            def _():
                wait_scatter(i - 2, o - 2)
        elif i >= 2:
            wait_scatter(i - 2, o - 2)
        if pf_ft:
            wait_idx(i + 2, o + 2)
            issue_fetch(i + 2, o + 2)
        if pf_ix:
            issue_idx(i + 5, o + 5)

        @plsc.parallel_loop(0, CE, 1, unroll=2)
        def _(r):
            for f in range(D // L):
                sl = pl.ds(f * L, L)
                xbuf[o % 4, r, sl] = jnp.maximum(
                    xbuf[o % 4, r, sl] + ebuf[o % 2, r, sl], 0.0)

        issue_scatter(i, o)
        if pf_ft:
            issue_emsg(i + 2, o + 2)

    for j in range(4):
        issue_idx(j, j)
    wait_idx(0, 0)
    issue_fetch(0, 0)
    issue_emsg(0, 0)
    issue_idx(4, 4)
    wait_idx(1, 1)
    issue_fetch(1, 1)
    issue_emsg(1, 1)

    def oct8(k, carry):
        i = 8 * k
        for o in range(8):
            step(i + o, o, True, True)
        return carry

    _DS = 8 * (NCHUNK // 8 - 1)
    lax.fori_loop(0, NCHUNK // 8 - 1, oct8, 0)
    for i in range(_DS, NCHUNK):
        step(i, i, i + 2 <= NCHUNK - 1, i + 5 <= NCHUNK - 1,
             guarded=False)
    wait_scatter(NCHUNK - 2, NCHUNK - 2)
    wait_scatter(NCHUNK - 1, NCHUNK - 1)

    plsc.subcore_barrier()
    pltpu.sync_copy(agg.at[pl.ds(s * ROWS_PT, ROWS_PT), :],
                    out_hbm.at[c, pl.ds(s * ROWS_PT, ROWS_PT), :])


_NBLK = 8
_NROWS = NP // _NBLK


def _mlp_body(x, a0, a1, bgrp, Wa, ba, g, bt, Wb, bb, out, p):
    h = x[:] + a0[:] + a1[:]
    h = jnp.dot(h, Wa[:], preferred_element_type=jnp.float32) + ba[:]
    h = h * (g[:] * lax.rsqrt(jnp.float32(1.0 + BN_EPS))) + bt[:]
    h = jnp.maximum(h, 0.0)
    h = jnp.dot(h, Wb[:], preferred_element_type=jnp.float32) + bb[:]
    h = jnp.maximum(h, 0.0)
    out[:] = h
    brow = bgrp[0, 0:1, :]
    gids = lax.broadcasted_iota(jnp.int32, (G, _NROWS), 0)
    onehot = (gids == brow).astype(jnp.float32)
    pp = jnp.dot(onehot, h, preferred_element_type=jnp.float32)

    @pl.when(pl.program_id(0) == 0)
    def _():
        p[:] = jnp.zeros_like(p)

    p[:] += pp


def _mlp(xp, aggp, bgrp3, Wa, ba, g, bt, Wb, bb):
    rspec = pl.BlockSpec((_NROWS, D), lambda i: (i, 0))
    wspec = pl.BlockSpec((D, H), lambda i: (0, 0))
    vspec = pl.BlockSpec((1, H), lambda i: (0, 0))
    return pl.pallas_call(
        _mlp_body,
        grid=(_NBLK,),
        in_specs=[rspec, rspec, rspec,
                  pl.BlockSpec((1, 8, _NROWS), lambda i: (i, 0, 0)),
                  wspec, vspec, vspec, vspec, wspec, vspec],
        out_specs=[rspec, pl.BlockSpec((G, H), lambda i: (0, 0))],
        out_shape=[jax.ShapeDtypeStruct((NP, H), jnp.float32),
                   jax.ShapeDtypeStruct((G, H), jnp.float32)],
    )(xp, aggp[0], aggp[1], bgrp3, Wa, ba, g, bt, Wb, bb)


def _head_body(p1, p2, p3, Wl1, bl1, Wl2, bl2, out, ls):
    hcat = jnp.concatenate([p1[:], p2[:], p3[:]], axis=1)
    hh = jnp.dot(hcat, Wl1[:], preferred_element_type=jnp.float32) + bl1[:]
    hh = jnp.maximum(hh, 0.0)
    logits = jnp.dot(hh, Wl2[:], preferred_element_type=jnp.float32) + bl2[:]
    col = lax.broadcasted_iota(jnp.int32, (G, H), 1)
    masked = jnp.where(col < C, logits, jnp.float32(-1e30))
    m = jnp.max(masked, axis=1, keepdims=True)
    lse = m + jnp.log(jnp.sum(jnp.exp(masked - m), axis=1, keepdims=True))
    out[:] = logits
    ls[:] = masked - lse


def _head(p1, p2, p3, Wl1, bl1, Wl2p, bl2p):
    def full(shape):
        return pl.BlockSpec(shape, lambda: tuple(0 for _ in shape))
    pspec = full((G, H))
    return pl.pallas_call(
        _head_body,
        in_specs=[pspec, pspec, pspec, full((3 * H, 3 * H)),
                  full((1, 3 * H)), full((3 * H, H)), full((1, H))],
        out_specs=[full((G, H)), full((G, H))],
        out_shape=[jax.ShapeDtypeStruct((G, H), jnp.float32)] * 2,
    )(p1, p2, p3, Wl1, bl1, Wl2p, bl2p)


def kernel(x, edge_index, edge_attr, batch,
           We1, be1, W1a, b1a, g1, bt1, W1b, b1b,
           We2, be2, W2a, b2a, g2, bt2, W2b, b2b,
           We3, be3, W3a, b3a, g3, bt3, W3b, b3b,
           Wl1, bl1, Wl2, bl2):
    src, dst = edge_index[0], edge_index[1]
    r1 = lambda v: v.reshape(1, -1)
    zeros = jnp.zeros((NP, D), jnp.float32)
    xp = jnp.concatenate([x, jnp.zeros((NP - N, D), jnp.float32)], axis=0)
    idx2 = jnp.stack([src.reshape(NW, NCHUNK, CE),
                      dst.reshape(NW, NCHUNK, CE)], axis=2)
    bpad = jnp.concatenate([batch, jnp.full((NP - N,), -1, jnp.int32)])
    bgrp3 = jnp.tile(bpad.reshape(_NBLK, 1, _NROWS), (1, 8, 1))

    e1 = _edge_transform(edge_attr, We1, r1(be1))
    a1p = _sc_msg(xp, e1, idx2, zeros)
    e2 = _edge_transform(edge_attr, We2, r1(be2))
    h1, p1 = _mlp(xp, a1p, bgrp3, W1a, r1(b1a), r1(g1), r1(bt1), W1b, r1(b1b))
    a2p = _sc_msg(h1, e2, idx2, zeros)
    e3 = _edge_transform(edge_attr, We3, r1(be3))
    h2, p2 = _mlp(h1, a2p, bgrp3, W2a, r1(b2a), r1(g2), r1(bt2), W2b, r1(b2b))
    a3p = _sc_msg(h2, e3, idx2, zeros)
    h3, p3 = _mlp(h2, a3p, bgrp3, W3a, r1(b3a), r1(g3), r1(bt3), W3b, r1(b3b))

    Wl2p = jnp.concatenate([Wl2, jnp.zeros((3 * H, H - C), jnp.float32)], axis=1)
    bl2p = jnp.concatenate([bl2, jnp.zeros((H - C,), jnp.float32)])
    out_f, ls_f = _head(p1, p2, p3, Wl1, r1(bl1), Wl2p, r1(bl2p))
    return (out_f[:, :C], ls_f[:, :C])


# speedup vs baseline: 1.0396x; 1.0396x over previous
import functools

import jax
import jax.numpy as jnp
from jax import lax
from jax.experimental import pallas as pl
from jax.experimental.pallas import tpu as pltpu
from jax.experimental.pallas import tpu_sc as plsc

N, E, D, ED, H, C, G = 10000, 320000, 128, 16, 128, 10, 128
BN_EPS = 1e-5
NP = 10240
NC, NS, L = 2, 16, 16
NW = NC * NS
EPW = E // NW
CE = 40
NCHUNK = EPW // CE
ROWS_PT = NP // NS

_EBLK = 8000


def _edge_body(ea, We1, be1, We2, be2, We3, be3, e1, e2, e3):
    a = ea[:]
    e1[:] = jnp.dot(a, We1[:], preferred_element_type=jnp.float32) + be1[:]
    e2[:] = jnp.dot(a, We2[:], preferred_element_type=jnp.float32) + be2[:]
    e3[:] = jnp.dot(a, We3[:], preferred_element_type=jnp.float32) + be3[:]


def _edge_transform(ea, We1, be1, We2, be2, We3, be3):
    nblk = E // _EBLK
    wspec = pl.BlockSpec((ED, D), lambda i: (0, 0))
    bspec = pl.BlockSpec((1, D), lambda i: (0, 0))
    ospec = pl.BlockSpec((_EBLK, D), lambda i: (i, 0))
    return pl.pallas_call(
        _edge_body,
        grid=(nblk,),
        in_specs=[pl.BlockSpec((_EBLK, ED), lambda i: (i, 0)),
                  wspec, bspec, wspec, bspec, wspec, bspec],
        out_specs=[ospec, ospec, ospec],
        out_shape=[jax.ShapeDtypeStruct((E, D), jnp.float32)] * 3,
    )(ea, We1, be1, We2, be2, We3, be3)


_sc_mesh = plsc.VectorSubcoreMesh(
    core_axis_name="c", subcore_axis_name="s", num_cores=NC, num_subcores=NS)


@functools.partial(
    pl.kernel,
    out_type=jax.ShapeDtypeStruct((NC, NP, D), jnp.float32),
    mesh=_sc_mesh,
    scratch_types=[
        pltpu.VMEM((8, 2, CE), jnp.int32),
        pltpu.VMEM((4, CE, D), jnp.float32),
        pltpu.VMEM((2, CE, D), jnp.float32),
        pltpu.VMEM_SHARED((NP, D), jnp.float32),
        pltpu.SemaphoreType.DMA,
        pltpu.SemaphoreType.DMA,
        pltpu.SemaphoreType.DMA,
        pltpu.SemaphoreType.DMA,
        pltpu.SemaphoreType.DMA,
        pltpu.SemaphoreType.DMA,
        pltpu.SemaphoreType.DMA,
        pltpu.SemaphoreType.DMA,
        pltpu.SemaphoreType.DMA,
        pltpu.SemaphoreType.DMA,
        pltpu.SemaphoreType.DMA,
        pltpu.SemaphoreType.DMA,
        pltpu.SemaphoreType.DMA,
        pltpu.SemaphoreType.DMA,
    ],
)
def _sc_msg(x_hbm, e_hbm, idx_hbm, zero_hbm, out_hbm,
            idxb, xbuf, ebuf, agg,
            i0, i1, i2, i3, g0, g1, g2, g3, m0, m1, s0, s1, s2, s3):
    c = lax.axis_index("c")
    s = lax.axis_index("s")
    wid = s * NC + c
    isem = (i0, i1, i2, i3)
    gsem = (g0, g1, g2, g3)
    msem = (m0, m1)
    ssem = (s0, s1, s2, s3)

    @pl.when(s == 0)
    def _():
        pltpu.sync_copy(zero_hbm, agg)

    plsc.subcore_barrier()

    def issue_idx(j, o):
        pltpu.async_copy(idx_hbm.at[wid, j], idxb.at[o % 8], isem[o % 4])

    def wait_idx(j, o):
        pltpu.make_async_copy(idx_hbm.at[wid, j], idxb.at[o % 8],
                              isem[o % 4]).wait()

    def issue_fetch(j, o):
        pltpu.async_copy(x_hbm.at[idxb.at[o % 8, 0]], xbuf.at[o % 4],
                         gsem[o % 4])

    def issue_emsg(j, o):
        ebase = wid * EPW + j * CE
        pltpu.async_copy(e_hbm.at[pl.ds(ebase, CE), :], ebuf.at[o % 2],
                         msem[o % 2])

    def wait_fetch(j, o):
        pltpu.make_async_copy(x_hbm.at[idxb.at[o % 8, 0]], xbuf.at[o % 4],
                              gsem[o % 4]).wait()
        ebase = wid * EPW + j * CE
        pltpu.make_async_copy(e_hbm.at[pl.ds(ebase, CE), :], ebuf.at[o % 2],
                              msem[o % 2]).wait()

    def issue_scatter(j, o):
        pltpu.async_copy(xbuf.at[o % 4], agg.at[idxb.at[o % 8, 1]],
                         ssem[o % 4], add=True)

    def wait_scatter(j, o):
        pltpu.make_async_copy(xbuf.at[o % 4], agg.at[idxb.at[o % 8, 1]],
                              ssem[o % 4]).wait()

    def step(i, o, pf_ft, pf_ix, guarded=True):
        wait_fetch(i, o)
        if guarded:
            @pl.when(i >= 2)
            def _():
                wait_scatter(i - 2, o - 2)
        elif i >= 2:
            wait_scatter(i - 2, o - 2)
        if pf_ft:
            wait_idx(i + 2, o + 2)
            issue_fetch(i + 2, o + 2)
        if pf_ix:
            issue_idx(i + 5, o + 5)

        @plsc.parallel_loop(0, CE, 1, unroll=2)
        def _(r):
            for f in range(D // L):
                sl = pl.ds(f * L, L)
                xbuf[o % 4, r, sl] = jnp.maximum(
                    xbuf[o % 4, r, sl] + ebuf[o % 2, r, sl], 0.0)

        issue_scatter(i, o)
        if pf_ft:
            issue_emsg(i + 2, o + 2)

    for j in range(4):
        issue_idx(j, j)
    wait_idx(0, 0)
    issue_fetch(0, 0)
    issue_emsg(0, 0)
    issue_idx(4, 4)
    wait_idx(1, 1)
    issue_fetch(1, 1)
    issue_emsg(1, 1)

    def oct8(k, carry):
        i = 8 * k
        for o in range(8):
            step(i + o, o, True, True)
        return carry

    _DS = 8 * (NCHUNK // 8 - 1)
    lax.fori_loop(0, NCHUNK // 8 - 1, oct8, 0)
    for i in range(_DS, NCHUNK):
        step(i, i, i + 2 <= NCHUNK - 1, i + 5 <= NCHUNK - 1,
             guarded=False)
    wait_scatter(NCHUNK - 2, NCHUNK - 2)
    wait_scatter(NCHUNK - 1, NCHUNK - 1)

    plsc.subcore_barrier()
    pltpu.sync_copy(agg.at[pl.ds(s * ROWS_PT, ROWS_PT), :],
                    out_hbm.at[c, pl.ds(s * ROWS_PT, ROWS_PT), :])


_NBLK = 8
_NROWS = NP // _NBLK


def _mlp_body(x, a0, a1, bgrp, Wa, ba, g, bt, Wb, bb, out, p):
    h = x[:] + a0[:] + a1[:]
    h = jnp.dot(h, Wa[:], preferred_element_type=jnp.float32) + ba[:]
    h = h * (g[:] * lax.rsqrt(jnp.float32(1.0 + BN_EPS))) + bt[:]
    h = jnp.maximum(h, 0.0)
    h = jnp.dot(h, Wb[:], preferred_element_type=jnp.float32) + bb[:]
    h = jnp.maximum(h, 0.0)
    out[:] = h
    brow = bgrp[0, 0:1, :]
    gids = lax.broadcasted_iota(jnp.int32, (G, _NROWS), 0)
    onehot = (gids == brow).astype(jnp.float32)
    pp = jnp.dot(onehot, h, preferred_element_type=jnp.float32)

    @pl.when(pl.program_id(0) == 0)
    def _():
        p[:] = jnp.zeros_like(p)

    p[:] += pp


def _mlp(xp, aggp, bgrp3, Wa, ba, g, bt, Wb, bb):
    rspec = pl.BlockSpec((_NROWS, D), lambda i: (i, 0))
    wspec = pl.BlockSpec((D, H), lambda i: (0, 0))
    vspec = pl.BlockSpec((1, H), lambda i: (0, 0))
    return pl.pallas_call(
        _mlp_body,
        grid=(_NBLK,),
        in_specs=[rspec, rspec, rspec,
                  pl.BlockSpec((1, 8, _NROWS), lambda i: (i, 0, 0)),
                  wspec, vspec, vspec, vspec, wspec, vspec],
        out_specs=[rspec, pl.BlockSpec((G, H), lambda i: (0, 0))],
        out_shape=[jax.ShapeDtypeStruct((NP, H), jnp.float32),
                   jax.ShapeDtypeStruct((G, H), jnp.float32)],
    )(xp, aggp[0], aggp[1], bgrp3, Wa, ba, g, bt, Wb, bb)


def _head_body(p1, p2, p3, Wl1, bl1, Wl2, bl2, out, ls):
    hcat = jnp.concatenate([p1[:], p2[:], p3[:]], axis=1)
    hh = jnp.dot(hcat, Wl1[:], preferred_element_type=jnp.float32) + bl1[:]
    hh = jnp.maximum(hh, 0.0)
    logits = jnp.dot(hh, Wl2[:], preferred_element_type=jnp.float32) + bl2[:]
    col = lax.broadcasted_iota(jnp.int32, (G, H), 1)
    masked = jnp.where(col < C, logits, jnp.float32(-1e30))
    m = jnp.max(masked, axis=1, keepdims=True)
    lse = m + jnp.log(jnp.sum(jnp.exp(masked - m), axis=1, keepdims=True))
    out[:] = logits
    ls[:] = masked - lse


def _head(p1, p2, p3, Wl1, bl1, Wl2p, bl2p):
    def full(shape):
        return pl.BlockSpec(shape, lambda: tuple(0 for _ in shape))
    pspec = full((G, H))
    return pl.pallas_call(
        _head_body,
        in_specs=[pspec, pspec, pspec, full((3 * H, 3 * H)),
                  full((1, 3 * H)), full((3 * H, H)), full((1, H))],
        out_specs=[full((G, H)), full((G, H))],
        out_shape=[jax.ShapeDtypeStruct((G, H), jnp.float32)] * 2,
    )(p1, p2, p3, Wl1, bl1, Wl2p, bl2p)


def kernel(x, edge_index, edge_attr, batch,
           We1, be1, W1a, b1a, g1, bt1, W1b, b1b,
           We2, be2, W2a, b2a, g2, bt2, W2b, b2b,
           We3, be3, W3a, b3a, g3, bt3, W3b, b3b,
           Wl1, bl1, Wl2, bl2):
    src, dst = edge_index[0], edge_index[1]
    r1 = lambda v: v.reshape(1, -1)
    zeros = jnp.zeros((NP, D), jnp.float32)
    xp = jnp.concatenate([x, jnp.zeros((NP - N, D), jnp.float32)], axis=0)
    idx2 = jnp.stack([src.reshape(NW, NCHUNK, CE),
                      dst.reshape(NW, NCHUNK, CE)], axis=2)
    bpad = jnp.concatenate([batch, jnp.full((NP - N,), -1, jnp.int32)])
    bgrp3 = jnp.tile(bpad.reshape(_NBLK, 1, _NROWS), (1, 8, 1))

    e1, e2, e3 = _edge_transform(edge_attr, We1, r1(be1), We2, r1(be2),
                                 We3, r1(be3))
    a1p = _sc_msg(xp, e1, idx2, zeros)
    h1, p1 = _mlp(xp, a1p, bgrp3, W1a, r1(b1a), r1(g1), r1(bt1), W1b, r1(b1b))
    a2p = _sc_msg(h1, e2, idx2, zeros)
    h2, p2 = _mlp(h1, a2p, bgrp3, W2a, r1(b2a), r1(g2), r1(bt2), W2b, r1(b2b))
    a3p = _sc_msg(h2, e3, idx2, zeros)
    h3, p3 = _mlp(h2, a3p, bgrp3, W3a, r1(b3a), r1(g3), r1(bt3), W3b, r1(b3b))

    Wl2p = jnp.concatenate([Wl2, jnp.zeros((3 * H, H - C), jnp.float32)], axis=1)
    bl2p = jnp.concatenate([bl2, jnp.zeros((H - C,), jnp.float32)])
    out_f, ls_f = _head(p1, p2, p3, Wl1, r1(bl1), Wl2p, r1(bl2p))
    return (out_f[:, :C], ls_f[:, :C])


# speedup vs baseline: 1.0544x; 1.0142x over previous
import functools

import jax
import jax.numpy as jnp
from jax import lax
from jax.experimental import pallas as pl
from jax.experimental.pallas import tpu as pltpu
from jax.experimental.pallas import tpu_sc as plsc

N, E, D, ED, H, C, G = 10000, 320000, 128, 16, 128, 10, 128
BN_EPS = 1e-5
NP = 10240
NC, NS, L = 2, 16, 16
NW = NC * NS
EPW = E // NW
CE = 40
NCHUNK = EPW // CE
ROWS_PT = NP // NS

_EBLK = 8000


def _edge_body(ea, We1, be1, We2, be2, We3, be3, e1, e2, e3):
    a = ea[:]
    e1[:] = jnp.dot(a, We1[:], preferred_element_type=jnp.float32) + be1[:]
    e2[:] = jnp.dot(a, We2[:], preferred_element_type=jnp.float32) + be2[:]
    e3[:] = jnp.dot(a, We3[:], preferred_element_type=jnp.float32) + be3[:]


def _edge_transform(ea, We1, be1, We2, be2, We3, be3):
    nblk = E // _EBLK
    wspec = pl.BlockSpec((ED, D), lambda i: (0, 0))
    bspec = pl.BlockSpec((1, D), lambda i: (0, 0))
    ospec = pl.BlockSpec((_EBLK, D), lambda i: (i, 0))
    return pl.pallas_call(
        _edge_body,
        grid=(nblk,),
        in_specs=[pl.BlockSpec((_EBLK, ED), lambda i: (i, 0)),
                  wspec, bspec, wspec, bspec, wspec, bspec],
        out_specs=[ospec, ospec, ospec],
        out_shape=[jax.ShapeDtypeStruct((E, D), jnp.float32)] * 3,
    )(ea, We1, be1, We2, be2, We3, be3)


_sc_mesh = plsc.VectorSubcoreMesh(
    core_axis_name="c", subcore_axis_name="s", num_cores=NC, num_subcores=NS)


@functools.partial(
    pl.kernel,
    out_type=jax.ShapeDtypeStruct((NC, NP, D), jnp.float32),
    mesh=_sc_mesh,
    scratch_types=[
        pltpu.VMEM((8, 2, CE), jnp.int32),
        pltpu.VMEM((4, CE, D), jnp.float32),
        pltpu.VMEM((2, CE, D), jnp.float32),
        pltpu.VMEM_SHARED((NP, D), jnp.float32),
        pltpu.SemaphoreType.DMA,
        pltpu.SemaphoreType.DMA,
        pltpu.SemaphoreType.DMA,
        pltpu.SemaphoreType.DMA,
        pltpu.SemaphoreType.DMA,
        pltpu.SemaphoreType.DMA,
        pltpu.SemaphoreType.DMA,
        pltpu.SemaphoreType.DMA,
        pltpu.SemaphoreType.DMA,
        pltpu.SemaphoreType.DMA,
        pltpu.SemaphoreType.DMA,
        pltpu.SemaphoreType.DMA,
        pltpu.SemaphoreType.DMA,
        pltpu.SemaphoreType.DMA,
    ],
)
def _sc_msg(x_hbm, e_hbm, ei_hbm, zero_hbm, out_hbm,
            idxb, xbuf, ebuf, agg,
            i0, i1, i2, i3, g0, g1, g2, g3, m0, m1, s0, s1, s2, s3):
    c = lax.axis_index("c")
    s = lax.axis_index("s")
    wid = s * NC + c
    isem = (i0, i1, i2, i3)
    gsem = (g0, g1, g2, g3)
    msem = (m0, m1)
    ssem = (s0, s1, s2, s3)

    @pl.when(s == 0)
    def _():
        pltpu.sync_copy(zero_hbm, agg)

    plsc.subcore_barrier()

    def issue_idx(j, o):
        base = wid * EPW + j * CE
        pltpu.async_copy(ei_hbm.at[pl.ds(base, CE)], idxb.at[o % 8, 0],
                         isem[o % 4])
        pltpu.async_copy(ei_hbm.at[pl.ds(E + base, CE)], idxb.at[o % 8, 1],
                         isem[o % 4])

    def wait_idx(j, o):
        base = wid * EPW + j * CE
        pltpu.make_async_copy(ei_hbm.at[pl.ds(base, CE)],
                              idxb.at[o % 8, 0], isem[o % 4]).wait()
        pltpu.make_async_copy(ei_hbm.at[pl.ds(E + base, CE)],
                              idxb.at[o % 8, 1], isem[o % 4]).wait()

    def issue_fetch(j, o):
        pltpu.async_copy(x_hbm.at[idxb.at[o % 8, 0]], xbuf.at[o % 4],
                         gsem[o % 4])

    def issue_emsg(j, o):
        ebase = wid * EPW + j * CE
        pltpu.async_copy(e_hbm.at[pl.ds(ebase, CE), :], ebuf.at[o % 2],
                         msem[o % 2])

    def wait_fetch(j, o):
        pltpu.make_async_copy(x_hbm.at[idxb.at[o % 8, 0]], xbuf.at[o % 4],
                              gsem[o % 4]).wait()
        ebase = wid * EPW + j * CE
        pltpu.make_async_copy(e_hbm.at[pl.ds(ebase, CE), :], ebuf.at[o % 2],
                              msem[o % 2]).wait()

    def issue_scatter(j, o):
        pltpu.async_copy(xbuf.at[o % 4], agg.at[idxb.at[o % 8, 1]],
                         ssem[o % 4], add=True)

    def wait_scatter(j, o):
        pltpu.make_async_copy(xbuf.at[o % 4], agg.at[idxb.at[o % 8, 1]],
                              ssem[o % 4]).wait()

    def step(i, o, pf_ft, pf_ix, guarded=True):
        wait_fetch(i, o)
        if guarded:
            @pl.when(i >= 2)
            def _():
                wait_scatter(i - 2, o - 2)
        elif i >= 2:
            wait_scatter(i - 2, o - 2)
        if pf_ft:
            wait_idx(i + 2, o + 2)
            issue_fetch(i + 2, o + 2)
        if pf_ix:
            issue_idx(i + 5, o + 5)

        @plsc.parallel_loop(0, CE, 1, unroll=2)
        def _(r):
            for f in range(D // L):
                sl = pl.ds(f * L, L)
                xbuf[o % 4, r, sl] = jnp.maximum(
                    xbuf[o % 4, r, sl] + ebuf[o % 2, r, sl], 0.0)

        issue_scatter(i, o)
        if pf_ft:
            issue_emsg(i + 2, o + 2)

    for j in range(4):
        issue_idx(j, j)
    wait_idx(0, 0)
    issue_fetch(0, 0)
    issue_emsg(0, 0)
    issue_idx(4, 4)
    wait_idx(1, 1)
    issue_fetch(1, 1)
    issue_emsg(1, 1)

    def oct8(k, carry):
        i = 8 * k
        for o in range(8):
            step(i + o, o, True, True)
        return carry

    _DS = 8 * (NCHUNK // 8 - 1)
    lax.fori_loop(0, NCHUNK // 8 - 1, oct8, 0)
    for i in range(_DS, NCHUNK):
        step(i, i, i + 2 <= NCHUNK - 1, i + 5 <= NCHUNK - 1,
             guarded=False)
    wait_scatter(NCHUNK - 2, NCHUNK - 2)
    wait_scatter(NCHUNK - 1, NCHUNK - 1)

    plsc.subcore_barrier()
    pltpu.sync_copy(agg.at[pl.ds(s * ROWS_PT, ROWS_PT), :],
                    out_hbm.at[c, pl.ds(s * ROWS_PT, ROWS_PT), :])


_NBLK = 25
_NROWS = N // _NBLK


def _mlp_body(x, a0, a1, bgrp, Wa, ba, g, bt, Wb, bb, out, p):
    h = x[:] + a0[0] + a1[0]
    h = jnp.dot(h, Wa[:], preferred_element_type=jnp.float32) + ba[:]
    h = h * (g[:] * lax.rsqrt(jnp.float32(1.0 + BN_EPS))) + bt[:]
    h = jnp.maximum(h, 0.0)
    h = jnp.dot(h, Wb[:], preferred_element_type=jnp.float32) + bb[:]
    h = jnp.maximum(h, 0.0)
    out[:] = h
    brow = bgrp[0, 0:1, :]
    gids = lax.broadcasted_iota(jnp.int32, (G, _NROWS), 0)
    onehot = (gids == brow).astype(jnp.float32)
    pp = jnp.dot(onehot, h, preferred_element_type=jnp.float32)

    @pl.when(pl.program_id(0) == 0)
    def _():
        p[:] = jnp.zeros_like(p)

    p[:] += pp


def _mlp(xp, aggp, bgrp3, Wa, ba, g, bt, Wb, bb):
    rspec = pl.BlockSpec((_NROWS, D), lambda i: (i, 0))
    wspec = pl.BlockSpec((D, H), lambda i: (0, 0))
    vspec = pl.BlockSpec((1, H), lambda i: (0, 0))
    aspec = pl.BlockSpec((1, _NROWS, D), lambda i: (0, i, 0))
    return pl.pallas_call(
        _mlp_body,
        grid=(_NBLK,),
        in_specs=[rspec, aspec, aspec,
                  pl.BlockSpec((1, 8, _NROWS), lambda i: (i, 0, 0)),
                  wspec, vspec, vspec, vspec, wspec, vspec],
        out_specs=[rspec, pl.BlockSpec((G, H), lambda i: (0, 0))],
        out_shape=[jax.ShapeDtypeStruct((N, H), jnp.float32),
                   jax.ShapeDtypeStruct((G, H), jnp.float32)],
    )(xp, aggp[0:1], aggp[1:2], bgrp3, Wa, ba, g, bt, Wb, bb)


def _head_body(p1, p2, p3, Wl1, bl1, Wl2, bl2, out, ls):
    hcat = jnp.concatenate([p1[:], p2[:], p3[:]], axis=1)
    hh = jnp.dot(hcat, Wl1[:], preferred_element_type=jnp.float32) + bl1[:]
    hh = jnp.maximum(hh, 0.0)
    logits = jnp.dot(hh, Wl2[:], preferred_element_type=jnp.float32) + bl2[:]
    col = lax.broadcasted_iota(jnp.int32, (G, H), 1)
    masked = jnp.where(col < C, logits, jnp.float32(-1e30))
    m = jnp.max(masked, axis=1, keepdims=True)
    lse = m + jnp.log(jnp.sum(jnp.exp(masked - m), axis=1, keepdims=True))
    out[:] = logits
    ls[:] = masked - lse


def _head(p1, p2, p3, Wl1, bl1, Wl2p, bl2p):
    def full(shape):
        return pl.BlockSpec(shape, lambda: tuple(0 for _ in shape))
    pspec = full((G, H))
    return pl.pallas_call(
        _head_body,
        in_specs=[pspec, pspec, pspec, full((3 * H, 3 * H)),
                  full((1, 3 * H)), full((3 * H, H)), full((1, H))],
        out_specs=[full((G, H)), full((G, H))],
        out_shape=[jax.ShapeDtypeStruct((G, H), jnp.float32)] * 2,
    )(p1, p2, p3, Wl1, bl1, Wl2p, bl2p)


def kernel(x, edge_index, edge_attr, batch,
           We1, be1, W1a, b1a, g1, bt1, W1b, b1b,
           We2, be2, W2a, b2a, g2, bt2, W2b, b2b,
           We3, be3, W3a, b3a, g3, bt3, W3b, b3b,
           Wl1, bl1, Wl2, bl2):
    r1 = lambda v: v.reshape(1, -1)
    zeros = jnp.zeros((NP, D), jnp.float32)
    bgrp3 = jnp.tile(batch.reshape(_NBLK, 1, _NROWS), (1, 8, 1))

    e1, e2, e3 = _edge_transform(edge_attr, We1, r1(be1), We2, r1(be2),
                                 We3, r1(be3))
    ei_flat = edge_index.reshape(2 * E)
    a1p = _sc_msg(x, e1, ei_flat, zeros)
    h1, p1 = _mlp(x, a1p, bgrp3, W1a, r1(b1a), r1(g1), r1(bt1), W1b, r1(b1b))
    a2p = _sc_msg(h1, e2, ei_flat, zeros)
    h2, p2 = _mlp(h1, a2p, bgrp3, W2a, r1(b2a), r1(g2), r1(bt2), W2b, r1(b2b))
    a3p = _sc_msg(h2, e3, ei_flat, zeros)
    h3, p3 = _mlp(h2, a3p, bgrp3, W3a, r1(b3a), r1(g3), r1(bt3), W3b, r1(b3b))

    Wl2p = jnp.concatenate([Wl2, jnp.zeros((3 * H, H - C), jnp.float32)], axis=1)
    bl2p = jnp.concatenate([bl2, jnp.zeros((H - C,), jnp.float32)])
    out_f, ls_f = _head(p1, p2, p3, Wl1, r1(bl1), Wl2p, r1(bl2p))
    return (out_f[:, :C], ls_f[:, :C])


# speedup vs baseline: 1.0566x; 1.0021x over previous
import functools

import jax
import jax.numpy as jnp
from jax import lax
from jax.experimental import pallas as pl
from jax.experimental.pallas import tpu as pltpu
from jax.experimental.pallas import tpu_sc as plsc

N, E, D, ED, H, C, G = 10000, 320000, 128, 16, 128, 10, 128
BN_EPS = 1e-5
NP = 10240
NC, NS, L = 2, 16, 16
NW = NC * NS
EPW = E // NW
CE = 40
NCHUNK = EPW // CE
ROWS_PT = NP // NS

_EBLK = 10000


def _edge_body(ea, We1, be1, We2, be2, We3, be3, e1, e2, e3):
    a = ea[:]
    e1[:] = jnp.dot(a, We1[:], preferred_element_type=jnp.float32) + be1[:]
    e2[:] = jnp.dot(a, We2[:], preferred_element_type=jnp.float32) + be2[:]
    e3[:] = jnp.dot(a, We3[:], preferred_element_type=jnp.float32) + be3[:]


def _edge_transform(ea, We1, be1, We2, be2, We3, be3):
    nblk = E // _EBLK
    wspec = pl.BlockSpec((ED, D), lambda i: (0, 0))
    bspec = pl.BlockSpec((1, D), lambda i: (0, 0))
    ospec = pl.BlockSpec((_EBLK, D), lambda i: (i, 0))
    return pl.pallas_call(
        _edge_body,
        grid=(nblk,),
        in_specs=[pl.BlockSpec((_EBLK, ED), lambda i: (i, 0)),
                  wspec, bspec, wspec, bspec, wspec, bspec],
        out_specs=[ospec, ospec, ospec],
        out_shape=[jax.ShapeDtypeStruct((E, D), jnp.float32)] * 3,
    )(ea, We1, be1, We2, be2, We3, be3)


_sc_mesh = plsc.VectorSubcoreMesh(
    core_axis_name="c", subcore_axis_name="s", num_cores=NC, num_subcores=NS)


@functools.partial(
    pl.kernel,
    out_type=jax.ShapeDtypeStruct((NC, NP, D), jnp.float32),
    mesh=_sc_mesh,
    scratch_types=[
        pltpu.VMEM((8, 2, CE), jnp.int32),
        pltpu.VMEM((4, CE, D), jnp.float32),
        pltpu.VMEM((2, CE, D), jnp.float32),
        pltpu.VMEM_SHARED((NP, D), jnp.float32),
        pltpu.SemaphoreType.DMA,
        pltpu.SemaphoreType.DMA,
        pltpu.SemaphoreType.DMA,
        pltpu.SemaphoreType.DMA,
        pltpu.SemaphoreType.DMA,
        pltpu.SemaphoreType.DMA,
        pltpu.SemaphoreType.DMA,
        pltpu.SemaphoreType.DMA,
        pltpu.SemaphoreType.DMA,
        pltpu.SemaphoreType.DMA,
        pltpu.SemaphoreType.DMA,
        pltpu.SemaphoreType.DMA,
        pltpu.SemaphoreType.DMA,
        pltpu.SemaphoreType.DMA,
    ],
)
def _sc_msg(x_hbm, e_hbm, ei_hbm, zero_hbm, out_hbm,
            idxb, xbuf, ebuf, agg,
            i0, i1, i2, i3, g0, g1, g2, g3, m0, m1, s0, s1, s2, s3):
    c = lax.axis_index("c")
    s = lax.axis_index("s")
    wid = s * NC + c
    isem = (i0, i1, i2, i3)
    gsem = (g0, g1, g2, g3)
    msem = (m0, m1)
    ssem = (s0, s1, s2, s3)

    @pl.when(s == 0)
    def _():
        pltpu.sync_copy(zero_hbm, agg)

    plsc.subcore_barrier()

    def issue_idx(j, o):
        base = wid * EPW + j * CE
        pltpu.async_copy(ei_hbm.at[pl.ds(base, CE)], idxb.at[o % 8, 0],
                         isem[o % 4])
        pltpu.async_copy(ei_hbm.at[pl.ds(E + base, CE)], idxb.at[o % 8, 1],
                         isem[o % 4])

    def wait_idx(j, o):
        base = wid * EPW + j * CE
        pltpu.make_async_copy(ei_hbm.at[pl.ds(base, CE)],
                              idxb.at[o % 8, 0], isem[o % 4]).wait()
        pltpu.make_async_copy(ei_hbm.at[pl.ds(E + base, CE)],
                              idxb.at[o % 8, 1], isem[o % 4]).wait()

    def issue_fetch(j, o):
        pltpu.async_copy(x_hbm.at[idxb.at[o % 8, 0]], xbuf.at[o % 4],
                         gsem[o % 4])

    def issue_emsg(j, o):
        ebase = wid * EPW + j * CE
        pltpu.async_copy(e_hbm.at[pl.ds(ebase, CE), :], ebuf.at[o % 2],
                         msem[o % 2])

    def wait_fetch(j, o):
        pltpu.make_async_copy(x_hbm.at[idxb.at[o % 8, 0]], xbuf.at[o % 4],
                              gsem[o % 4]).wait()
        ebase = wid * EPW + j * CE
        pltpu.make_async_copy(e_hbm.at[pl.ds(ebase, CE), :], ebuf.at[o % 2],
                              msem[o % 2]).wait()

    def issue_scatter(j, o):
        pltpu.async_copy(xbuf.at[o % 4], agg.at[idxb.at[o % 8, 1]],
                         ssem[o % 4], add=True)

    def wait_scatter(j, o):
        pltpu.make_async_copy(xbuf.at[o % 4], agg.at[idxb.at[o % 8, 1]],
                              ssem[o % 4]).wait()

    def step(i, o, pf_ft, pf_ix, guarded=True):
        wait_fetch(i, o)
        if guarded:
            @pl.when(i >= 2)
            def _():
                wait_scatter(i - 2, o - 2)
        elif i >= 2:
            wait_scatter(i - 2, o - 2)
        if pf_ft:
            wait_idx(i + 2, o + 2)
            issue_fetch(i + 2, o + 2)
        if pf_ix:
            issue_idx(i + 5, o + 5)

        @plsc.parallel_loop(0, CE, 1, unroll=2)
        def _(r):
            for f in range(D // L):
                sl = pl.ds(f * L, L)
                xbuf[o % 4, r, sl] = jnp.maximum(
                    xbuf[o % 4, r, sl] + ebuf[o % 2, r, sl], 0.0)

        issue_scatter(i, o)
        if pf_ft:
            issue_emsg(i + 2, o + 2)

    for j in range(4):
        issue_idx(j, j)
    wait_idx(0, 0)
    issue_fetch(0, 0)
    issue_emsg(0, 0)
    issue_idx(4, 4)
    wait_idx(1, 1)
    issue_fetch(1, 1)
    issue_emsg(1, 1)

    def oct8(k, carry):
        i = 8 * k
        for o in range(8):
            step(i + o, o, True, True)
        return carry

    _DS = 8 * (NCHUNK // 8 - 1)
    lax.fori_loop(0, NCHUNK // 8 - 1, oct8, 0)
    for i in range(_DS, NCHUNK):
        step(i, i, i + 2 <= NCHUNK - 1, i + 5 <= NCHUNK - 1,
             guarded=False)
    wait_scatter(NCHUNK - 2, NCHUNK - 2)
    wait_scatter(NCHUNK - 1, NCHUNK - 1)

    plsc.subcore_barrier()
    pltpu.sync_copy(agg.at[pl.ds(s * ROWS_PT, ROWS_PT), :],
                    out_hbm.at[c, pl.ds(s * ROWS_PT, ROWS_PT), :])


_NBLK = 25
_NROWS = N // _NBLK


def _mlp_body(x, a0, a1, bgrp, Wa, ba, g, bt, Wb, bb, out, p):
    h = x[:] + a0[0] + a1[0]
    h = jnp.dot(h, Wa[:], preferred_element_type=jnp.float32) + ba[:]
    h = h * (g[:] * lax.rsqrt(jnp.float32(1.0 + BN_EPS))) + bt[:]
    h = jnp.maximum(h, 0.0)
    h = jnp.dot(h, Wb[:], preferred_element_type=jnp.float32) + bb[:]
    h = jnp.maximum(h, 0.0)
    out[:] = h
    brow = bgrp[0, 0:1, :]
    gids = lax.broadcasted_iota(jnp.int32, (G, _NROWS), 0)
    onehot = (gids == brow).astype(jnp.float32)
    pp = jnp.dot(onehot, h, preferred_element_type=jnp.float32)

    @pl.when(pl.program_id(0) == 0)
    def _():
        p[:] = jnp.zeros_like(p)

    p[:] += pp


def _mlp(xp, aggp, bgrp3, Wa, ba, g, bt, Wb, bb):
    rspec = pl.BlockSpec((_NROWS, D), lambda i: (i, 0))
    wspec = pl.BlockSpec((D, H), lambda i: (0, 0))
    vspec = pl.BlockSpec((1, H), lambda i: (0, 0))
    aspec = pl.BlockSpec((1, _NROWS, D), lambda i: (0, i, 0))
    return pl.pallas_call(
        _mlp_body,
        grid=(_NBLK,),
        in_specs=[rspec, aspec, aspec,
                  pl.BlockSpec((1, 8, _NROWS), lambda i: (i, 0, 0)),
                  wspec, vspec, vspec, vspec, wspec, vspec],
        out_specs=[rspec, pl.BlockSpec((G, H), lambda i: (0, 0))],
        out_shape=[jax.ShapeDtypeStruct((N, H), jnp.float32),
                   jax.ShapeDtypeStruct((G, H), jnp.float32)],
    )(xp, aggp[0:1], aggp[1:2], bgrp3, Wa, ba, g, bt, Wb, bb)


def _head_body(p1, p2, p3, Wl1, bl1, Wl2, bl2, out, ls):
    hcat = jnp.concatenate([p1[:], p2[:], p3[:]], axis=1)
    hh = jnp.dot(hcat, Wl1[:], preferred_element_type=jnp.float32) + bl1[:]
    hh = jnp.maximum(hh, 0.0)
    logits = jnp.dot(hh, Wl2[:], preferred_element_type=jnp.float32) + bl2[:]
    col = lax.broadcasted_iota(jnp.int32, (G, H), 1)
    masked = jnp.where(col < C, logits, jnp.float32(-1e30))
    m = jnp.max(masked, axis=1, keepdims=True)
    lse = m + jnp.log(jnp.sum(jnp.exp(masked - m), axis=1, keepdims=True))
    out[:] = logits
    ls[:] = masked - lse


def _head(p1, p2, p3, Wl1, bl1, Wl2p, bl2p):
    def full(shape):
        return pl.BlockSpec(shape, lambda: tuple(0 for _ in shape))
    pspec = full((G, H))
    return pl.pallas_call(
        _head_body,
        in_specs=[pspec, pspec, pspec, full((3 * H, 3 * H)),
                  full((1, 3 * H)), full((3 * H, H)), full((1, H))],
        out_specs=[full((G, H)), full((G, H))],
        out_shape=[jax.ShapeDtypeStruct((G, H), jnp.float32)] * 2,
    )(p1, p2, p3, Wl1, bl1, Wl2p, bl2p)


def kernel(x, edge_index, edge_attr, batch,
           We1, be1, W1a, b1a, g1, bt1, W1b, b1b,
           We2, be2, W2a, b2a, g2, bt2, W2b, b2b,
           We3, be3, W3a, b3a, g3, bt3, W3b, b3b,
           Wl1, bl1, Wl2, bl2):
    r1 = lambda v: v.reshape(1, -1)
    zeros = jnp.zeros((NP, D), jnp.float32)
    bgrp3 = jnp.tile(batch.reshape(_NBLK, 1, _NROWS), (1, 8, 1))

    e1, e2, e3 = _edge_transform(edge_attr, We1, r1(be1), We2, r1(be2),
                                 We3, r1(be3))
    ei_flat = edge_index.reshape(2 * E)
    a1p = _sc_msg(x, e1, ei_flat, zeros)
    h1, p1 = _mlp(x, a1p, bgrp3, W1a, r1(b1a), r1(g1), r1(bt1), W1b, r1(b1b))
    a2p = _sc_msg(h1, e2, ei_flat, zeros)
    h2, p2 = _mlp(h1, a2p, bgrp3, W2a, r1(b2a), r1(g2), r1(bt2), W2b, r1(b2b))
    a3p = _sc_msg(h2, e3, ei_flat, zeros)
    h3, p3 = _mlp(h2, a3p, bgrp3, W3a, r1(b3a), r1(g3), r1(bt3), W3b, r1(b3b))

    Wl2p = jnp.concatenate([Wl2, jnp.zeros((3 * H, H - C), jnp.float32)], axis=1)
    bl2p = jnp.concatenate([bl2, jnp.zeros((H - C,), jnp.float32)])
    out_f, ls_f = _head(p1, p2, p3, Wl1, r1(bl1), Wl2p, r1(bl2p))
    return (out_f[:, :C], ls_f[:, :C])
